# baseline (speedup 1.0000x reference)
import functools

import jax
import jax.numpy as jnp
from jax import lax
from jax.experimental import pallas as pl
from jax.experimental.pallas import tpu as pltpu
from jax.experimental.pallas import tpu_sc as plsc

_NUM_CORES = 2
_NUM_SUBCORES = 16
_NUM_WORKERS = _NUM_CORES * _NUM_SUBCORES
_LANES = 16
_CHUNK = 128


@functools.lru_cache(maxsize=None)
def _make_gather(batch: int, dim: int):
    b_per_w = batch // _NUM_WORKERS
    n_chunks = b_per_w // _CHUNK
    mesh = plsc.VectorSubcoreMesh(core_axis_name="c", subcore_axis_name="s")

    @functools.partial(
        pl.kernel,
        mesh=mesh,
        out_type=jax.ShapeDtypeStruct((batch, 128), jnp.float32),
        compiler_params=pltpu.CompilerParams(needs_layout_passes=False),
        scratch_types=[
            pltpu.VMEM((b_per_w,), jnp.int32),
            pltpu.VMEM((n_chunks, _CHUNK), jnp.int32),
            pltpu.VMEM((b_per_w, 128), jnp.float32),
            pltpu.SemaphoreType.DMA,
        ],
    )
    def gather_kernel(pairs_hbm, idx_hbm, out_hbm, idx_v, pidx_v, rows_v,
                      sem):
        wid = lax.axis_index("s") * _NUM_CORES + lax.axis_index("c")
        base = wid * b_per_w
        iota = lax.iota(jnp.int32, _LANES)
        pltpu.sync_copy(idx_hbm.at[pl.ds(base, b_per_w)], idx_v)

        def pair_body(k, _):
            v = idx_v[pl.ds(k * _LANES, _LANES)]
            pidx_v[k // (_CHUNK // _LANES),
                   pl.ds((k % (_CHUNK // _LANES)) * _LANES, _LANES)] = v >> 1
            return ()

        lax.fori_loop(0, b_per_w // _LANES, pair_body, (), unroll=False)
        copies = []
        for j in range(n_chunks):
            copies.append(
                pltpu.async_copy(
                    pairs_hbm.at[pidx_v.at[j]],
                    rows_v.at[pl.ds(j * _CHUNK, _CHUNK)],
                    sem,
                )
            )
        for c in copies:
            c.wait()

        def sel_body(k, _):
            row = jnp.zeros((_LANES,), jnp.int32) + k
            hk = plsc.load_gather(idx_v, [row]) & 1
            for c0 in range(dim // _LANES):
                cols = hk * dim + c0 * _LANES + iota
                vals = plsc.load_gather(rows_v, [row, cols])
                rows_v[k, pl.ds(c0 * _LANES, _LANES)] = vals
            return ()

        lax.fori_loop(0, b_per_w, sel_body, (), unroll=False)
        pltpu.sync_copy(rows_v, out_hbm.at[pl.ds(base, b_per_w)])

    return gather_kernel


def kernel(input_x, Emb):
    batch = input_x.shape[1]
    n, dim = Emb.shape
    pairs = Emb.reshape(n // 2, 2 * dim)
    idx = input_x.reshape(batch)
    out = _make_gather(batch, dim)(pairs, idx)
    return out[:, :dim]
